# baseline (speedup 1.0000x reference)
import functools

import jax
import jax.numpy as jnp
from jax import lax
from jax.experimental import pallas as pl
from jax.experimental.pallas import tpu as pltpu
from jax.experimental.pallas import tpu_sc as plsc

B = 16384
NC, NS = 2, 16
NW = NC * NS
CAT_D = 64
ROWS_PER_W = B // NW
GROUP = 32

BLK = 512
GRID = B // BLK


def _sc_gather_body(idx_hbm, table_hbm, out_hbm, idx_v, rows_v, sem):
    wid = lax.axis_index("s") * NC + lax.axis_index("c")
    pltpu.sync_copy(idx_hbm.at[wid], idx_v)

    def issue(g):
        base = g * GROUP
        for k in range(GROUP // 16):
            ids16 = idx_v[pl.ds(base + k * 16, 16)]
            t16 = lax.shift_right_logical(ids16, 3)
            r16 = lax.bitwise_and(ids16, 7)
            for j in range(16):
                pltpu.async_copy(
                    table_hbm.at[t16[j], r16[j]],
                    rows_v.at[base + k * 16 + j], sem)

    def drain():
        for j in range(GROUP):
            pltpu.make_async_copy(table_hbm.at[0, 0], rows_v.at[0], sem).wait()

    issue(0)

    def group(g, carry):
        issue(g)
        drain()
        return carry

    lax.fori_loop(1, ROWS_PER_W // GROUP, group, 0)
    drain()
    pltpu.sync_copy(rows_v, out_hbm.at[pl.ds(wid * ROWS_PER_W, ROWS_PER_W)])


def _sc_gather(idx2d, table3):
    gather = pl.kernel(
        _sc_gather_body,
        out_type=jax.ShapeDtypeStruct((B, CAT_D), jnp.float32),
        mesh=plsc.VectorSubcoreMesh(
            core_axis_name="c", subcore_axis_name="s",
            num_cores=NC, num_subcores=NS,
        ),
        scratch_types=[
            pltpu.VMEM((ROWS_PER_W,), jnp.int32),
            pltpu.VMEM((ROWS_PER_W, CAT_D), jnp.float32),
            pltpu.SemaphoreType.DMA,
        ],
    )
    return gather(idx2d, table3)


def _prep_body(charp, pc_w1, pos, pc_b1, starp, attr_w, mbig, c0, a1):
    cp = charp[...]
    w1 = pc_w1[...]
    parts = [
        jnp.dot(cp, w1[32 * j:32 * (j + 1), :], preferred_element_type=jnp.float32)
        for j in range(10)
    ]
    mbig[...] = jnp.concatenate(parts, axis=0)
    acc = pc_b1[...]
    for j in range(10):
        acc = acc + jnp.dot(
            pos[j:j + 1, :], w1[32 * j:32 * (j + 1), :],
            preferred_element_type=jnp.float32,
        )
    c0[...] = acc
    a1[...] = jnp.dot(starp[...], attr_w[:16, :], preferred_element_type=jnp.float32)


def _ln(x, g, b, eps=1e-5):
    m = jnp.mean(x, axis=-1, keepdims=True)
    v = jnp.var(x, axis=-1, keepdims=True)
    return (x - m) / jnp.sqrt(v + eps) * g + b


_TDN = (((0,), (0,)), ((), ()))


def _tdot(lhs_t, rhs):
    return lax.dot_general(lhs_t, rhs, _TDN, preferred_element_type=jnp.float32)


def _tc1_body(pc_idx_t, temporal_t, star1_t, numerical_t,
              mbig, c0, pc_ln1_g, pc_ln1_b, pc_w2, pc_b2, pc_ln2_g, pc_ln2_b,
              tmp_w1, tmp_b1, tmp_w2, tmp_b2,
              a1, num_w1, num_b1, num_w2, num_b2, attr_w, attr_b,
              out_ref):
    f32 = jnp.float32
    idx_t = pc_idx_t[...]
    iota32 = lax.broadcasted_iota(jnp.int32, (32, BLK), 0)
    ohs_t = [(idx_t[j:j + 1, :] == iota32).astype(f32) for j in range(10)]
    oh_t = jnp.concatenate(ohs_t, axis=0)
    h = _tdot(oh_t, mbig[...]) + c0[...]
    h = jax.nn.gelu(_ln(h, pc_ln1_g[...], pc_ln1_b[...]))
    sp = _ln(jnp.dot(h, pc_w2[...], preferred_element_type=f32) + pc_b2[...],
             pc_ln2_g[...], pc_ln2_b[...])

    t = jax.nn.relu(_tdot(temporal_t[...], tmp_w1[...]) + tmp_b1[...])
    tmp_emb = jnp.dot(t, tmp_w2[...], preferred_element_type=f32) + tmp_b2[...]

    iota16 = lax.broadcasted_iota(jnp.int32, (16, BLK), 0)
    oh_star_t = (star1_t[...] == iota16).astype(f32)
    n = jax.nn.relu(_tdot(numerical_t[...], num_w1[...]) + num_b1[...])
    n = jnp.dot(n, num_w2[...], preferred_element_type=f32) + num_b2[...]
    attr = (_tdot(oh_star_t, a1[...])
            + jnp.dot(n, attr_w[16:32, :], preferred_element_type=f32)
            + attr_b[...])
    out_ref[...] = jnp.concatenate([sp, tmp_emb, attr], axis=-1)


def _ln_mxu(x, g, b, eps=1e-5):
    f32 = jnp.float32
    w = x.shape[-1]
    ones = jnp.ones((w, 1), f32)
    s1 = jnp.dot(x, ones, preferred_element_type=f32)
    s2 = jnp.dot(x * x, ones, preferred_element_type=f32)
    m = s1 * (1.0 / w)
    v = s2 * (1.0 / w) - m * m
    return (x - m) * lax.rsqrt(v + eps) * g + b


def _tc2_body(cat_emb, rest,
              fus_w1, fus_b1, fus_ln1_g, fus_ln1_b,
              fus_w2, fus_b2, fus_ln2_g, fus_ln2_b,
              out_ref):
    f32 = jnp.float32
    comb = jnp.concatenate([cat_emb[...], rest[...]], axis=-1)
    z = jnp.dot(comb, fus_w1[...], preferred_element_type=f32) + fus_b1[...]
    z = jax.nn.gelu(_ln_mxu(z, fus_ln1_g[...], fus_ln1_b[...]))
    o = jnp.dot(z, fus_w2[...], preferred_element_type=f32) + fus_b2[...]
    out_ref[...] = _ln_mxu(o, fus_ln2_g[...], fus_ln2_b[...])


def _row_spec(d):
    return pl.BlockSpec((BLK, d), lambda i: (i, 0))


def _full_spec(shape):
    return pl.BlockSpec(shape, lambda i: tuple(0 for _ in shape))


def kernel(category_ids, pluscode_indices, temporal_features, star_idx,
           numerical_features, cat_table, char_table, pos_table,
           pc_w1, pc_b1, pc_ln1_g, pc_ln1_b, pc_w2, pc_b2, pc_ln2_g, pc_ln2_b,
           tmp_w1, tmp_b1, tmp_w2, tmp_b2,
           star_table, num_w1, num_b1, num_w2, num_b2, attr_w, attr_b,
           fus_w1, fus_b1, fus_ln1_g, fus_ln1_b, fus_w2, fus_b2,
           fus_ln2_g, fus_ln2_b):
    f32 = jnp.float32

    ids32 = category_ids.astype(jnp.int32)
    idx2d = ids32.reshape(NW, ROWS_PER_W)
    table3 = cat_table.reshape(cat_table.shape[0] // 8, 8, CAT_D)

    charp = jnp.zeros((32, 32), f32).at[:22, :].set(char_table)
    starp = jnp.zeros((16, 16), f32).at[:11, :].set(star_table)

    mbig, c0, a1 = pl.pallas_call(
        _prep_body,
        out_shape=[
            jax.ShapeDtypeStruct((320, 128), f32),
            jax.ShapeDtypeStruct((1, 128), f32),
            jax.ShapeDtypeStruct((16, 32), f32),
        ],
    )(charp, pc_w1, pos_table, pc_b1.reshape(1, 128), starp, attr_w)

    tc1_row_in = [
        pl.BlockSpec((10, BLK), lambda i: (0, i)),
        pl.BlockSpec((6, BLK), lambda i: (0, i)),
        pl.BlockSpec((1, BLK), lambda i: (0, i)),
        pl.BlockSpec((3, BLK), lambda i: (0, i)),
    ]
    tc1_bcast = [
        mbig, c0, pc_ln1_g.reshape(1, 128), pc_ln1_b.reshape(1, 128),
        pc_w2, pc_b2.reshape(1, 64), pc_ln2_g.reshape(1, 64), pc_ln2_b.reshape(1, 64),
        tmp_w1, tmp_b1.reshape(1, 64), tmp_w2, tmp_b2.reshape(1, 32),
        a1, num_w1, num_b1.reshape(1, 32), num_w2, num_b2.reshape(1, 16),
        attr_w, attr_b.reshape(1, 32),
    ]
    rest = pl.pallas_call(
        _tc1_body,
        grid=(GRID,),
        in_specs=tc1_row_in + [_full_spec(a.shape) for a in tc1_bcast],
        out_specs=pl.BlockSpec((BLK, 128), lambda i: (i, 0)),
        out_shape=jax.ShapeDtypeStruct((B, 128), f32),
    )(
        pluscode_indices.astype(jnp.int32).T,
        temporal_features.T,
        star_idx.astype(jnp.int32).reshape(1, B),
        numerical_features.T,
        *tc1_bcast,
    )

    table3b, rest = jax.lax.optimization_barrier((table3, rest))
    cat_emb = _sc_gather(idx2d, table3b)

    tc2_bcast = [
        fus_w1, fus_b1.reshape(1, 512), fus_ln1_g.reshape(1, 512), fus_ln1_b.reshape(1, 512),
        fus_w2, fus_b2.reshape(1, 256), fus_ln2_g.reshape(1, 256), fus_ln2_b.reshape(1, 256),
    ]
    out = pl.pallas_call(
        _tc2_body,
        grid=(GRID,),
        in_specs=[_row_spec(CAT_D), _row_spec(128)]
        + [_full_spec(a.shape) for a in tc2_bcast],
        out_specs=pl.BlockSpec((BLK, 256), lambda i: (i, 0)),
        out_shape=jax.ShapeDtypeStruct((B, 256), f32),
    )(cat_emb, rest, *tc2_bcast)
    return out


# speedup vs baseline: 1.0328x; 1.0328x over previous
import functools

import jax
import jax.numpy as jnp
from jax import lax
from jax.experimental import pallas as pl
from jax.experimental.pallas import tpu as pltpu
from jax.experimental.pallas import tpu_sc as plsc

B = 16384
NC, NS = 2, 16
NW = NC * NS
CAT_D = 64
ROWS_PER_W = B // NW
GROUP = 32

BLK = 1024
GRID = B // BLK


def _sc_gather_body(idx_hbm, table_hbm, out_hbm, idx_v, rows_v, sem):
    wid = lax.axis_index("s") * NC + lax.axis_index("c")
    pltpu.sync_copy(idx_hbm.at[wid], idx_v)

    def issue(g):
        base = g * GROUP
        for k in range(GROUP // 16):
            ids16 = idx_v[pl.ds(base + k * 16, 16)]
            t16 = lax.shift_right_logical(ids16, 3)
            r16 = lax.bitwise_and(ids16, 7)
            for j in range(16):
                pltpu.async_copy(
                    table_hbm.at[t16[j], r16[j]],
                    rows_v.at[base + k * 16 + j], sem)

    def drain():
        for j in range(GROUP):
            pltpu.make_async_copy(table_hbm.at[0, 0], rows_v.at[0], sem).wait()

    issue(0)

    def group(g, carry):
        issue(g)
        drain()
        return carry

    lax.fori_loop(1, ROWS_PER_W // GROUP, group, 0)
    drain()
    pltpu.sync_copy(rows_v, out_hbm.at[pl.ds(wid * ROWS_PER_W, ROWS_PER_W)])


def _sc_gather(idx2d, table3):
    gather = pl.kernel(
        _sc_gather_body,
        out_type=jax.ShapeDtypeStruct((B, CAT_D), jnp.float32),
        mesh=plsc.VectorSubcoreMesh(
            core_axis_name="c", subcore_axis_name="s",
            num_cores=NC, num_subcores=NS,
        ),
        scratch_types=[
            pltpu.VMEM((ROWS_PER_W,), jnp.int32),
            pltpu.VMEM((ROWS_PER_W, CAT_D), jnp.float32),
            pltpu.SemaphoreType.DMA,
        ],
    )
    return gather(idx2d, table3)


def _prep_body(charp, pc_w1, pos, pc_b1, starp, attr_w, mbig, c0, a1):
    cp = charp[...]
    w1 = pc_w1[...]
    parts = [
        jnp.dot(cp, w1[32 * j:32 * (j + 1), :], preferred_element_type=jnp.float32)
        for j in range(10)
    ]
    mbig[...] = jnp.concatenate(parts, axis=0)
    acc = pc_b1[...]
    for j in range(10):
        acc = acc + jnp.dot(
            pos[j:j + 1, :], w1[32 * j:32 * (j + 1), :],
            preferred_element_type=jnp.float32,
        )
    c0[...] = acc
    a1[...] = jnp.dot(starp[...], attr_w[:16, :], preferred_element_type=jnp.float32)


def _ln(x, g, b, eps=1e-5):
    m = jnp.mean(x, axis=-1, keepdims=True)
    v = jnp.var(x, axis=-1, keepdims=True)
    return (x - m) / jnp.sqrt(v + eps) * g + b


_TDN = (((0,), (0,)), ((), ()))


def _tdot(lhs_t, rhs):
    return lax.dot_general(lhs_t, rhs, _TDN, preferred_element_type=jnp.float32)


def _tc1_body(pc_idx_t, temporal_t, star1_t, numerical_t,
              mbig, c0, pc_ln1_g, pc_ln1_b, pc_w2, pc_b2, pc_ln2_g, pc_ln2_b,
              tmp_w1, tmp_b1, tmp_w2, tmp_b2,
              a1, num_w1, num_b1, num_w2, num_b2, attr_w, attr_b,
              out_ref):
    f32 = jnp.float32
    idx_t = pc_idx_t[...]
    iota32 = lax.broadcasted_iota(jnp.int32, (32, BLK), 0)
    ohs_t = [(idx_t[j:j + 1, :] == iota32).astype(f32) for j in range(10)]
    oh_t = jnp.concatenate(ohs_t, axis=0)
    h = _tdot(oh_t, mbig[...]) + c0[...]
    h = jax.nn.gelu(_ln(h, pc_ln1_g[...], pc_ln1_b[...]))
    sp = _ln(jnp.dot(h, pc_w2[...], preferred_element_type=f32) + pc_b2[...],
             pc_ln2_g[...], pc_ln2_b[...])

    t = jax.nn.relu(_tdot(temporal_t[...], tmp_w1[...]) + tmp_b1[...])
    tmp_emb = jnp.dot(t, tmp_w2[...], preferred_element_type=f32) + tmp_b2[...]

    iota16 = lax.broadcasted_iota(jnp.int32, (16, BLK), 0)
    oh_star_t = (star1_t[...] == iota16).astype(f32)
    n = jax.nn.relu(_tdot(numerical_t[...], num_w1[...]) + num_b1[...])
    n = jnp.dot(n, num_w2[...], preferred_element_type=f32) + num_b2[...]
    attr = (_tdot(oh_star_t, a1[...])
            + jnp.dot(n, attr_w[16:32, :], preferred_element_type=f32)
            + attr_b[...])
    out_ref[...] = jnp.concatenate([sp, tmp_emb, attr], axis=-1)


def _ln_mxu(x, g, b, eps=1e-5):
    f32 = jnp.float32
    w = x.shape[-1]
    ones = jnp.ones((w, 1), f32)
    s1 = jnp.dot(x, ones, preferred_element_type=f32)
    s2 = jnp.dot(x * x, ones, preferred_element_type=f32)
    m = s1 * (1.0 / w)
    v = s2 * (1.0 / w) - m * m
    return (x - m) * lax.rsqrt(v + eps) * g + b


def _tc2_body(cat_emb, rest,
              fus_w1, fus_b1, fus_ln1_g, fus_ln1_b,
              fus_w2, fus_b2, fus_ln2_g, fus_ln2_b,
              out_ref):
    f32 = jnp.float32
    comb = jnp.concatenate([cat_emb[...], rest[...]], axis=-1)
    z = jnp.dot(comb, fus_w1[...], preferred_element_type=f32) + fus_b1[...]
    z = jax.nn.gelu(_ln(z, fus_ln1_g[...], fus_ln1_b[...]))
    o = jnp.dot(z, fus_w2[...], preferred_element_type=f32) + fus_b2[...]
    out_ref[...] = _ln(o, fus_ln2_g[...], fus_ln2_b[...])


def _row_spec(d):
    return pl.BlockSpec((BLK, d), lambda i: (i, 0))


def _full_spec(shape):
    return pl.BlockSpec(shape, lambda i: tuple(0 for _ in shape))


def kernel(category_ids, pluscode_indices, temporal_features, star_idx,
           numerical_features, cat_table, char_table, pos_table,
           pc_w1, pc_b1, pc_ln1_g, pc_ln1_b, pc_w2, pc_b2, pc_ln2_g, pc_ln2_b,
           tmp_w1, tmp_b1, tmp_w2, tmp_b2,
           star_table, num_w1, num_b1, num_w2, num_b2, attr_w, attr_b,
           fus_w1, fus_b1, fus_ln1_g, fus_ln1_b, fus_w2, fus_b2,
           fus_ln2_g, fus_ln2_b):
    f32 = jnp.float32

    ids32 = category_ids.astype(jnp.int32)
    idx2d = ids32.reshape(NW, ROWS_PER_W)
    table3 = cat_table.reshape(cat_table.shape[0] // 8, 8, CAT_D)

    charp = jnp.zeros((32, 32), f32).at[:22, :].set(char_table)
    starp = jnp.zeros((16, 16), f32).at[:11, :].set(star_table)

    mbig, c0, a1 = pl.pallas_call(
        _prep_body,
        out_shape=[
            jax.ShapeDtypeStruct((320, 128), f32),
            jax.ShapeDtypeStruct((1, 128), f32),
            jax.ShapeDtypeStruct((16, 32), f32),
        ],
    )(charp, pc_w1, pos_table, pc_b1.reshape(1, 128), starp, attr_w)

    tc1_row_in = [
        pl.BlockSpec((10, BLK), lambda i: (0, i)),
        pl.BlockSpec((6, BLK), lambda i: (0, i)),
        pl.BlockSpec((1, BLK), lambda i: (0, i)),
        pl.BlockSpec((3, BLK), lambda i: (0, i)),
    ]
    tc1_bcast = [
        mbig, c0, pc_ln1_g.reshape(1, 128), pc_ln1_b.reshape(1, 128),
        pc_w2, pc_b2.reshape(1, 64), pc_ln2_g.reshape(1, 64), pc_ln2_b.reshape(1, 64),
        tmp_w1, tmp_b1.reshape(1, 64), tmp_w2, tmp_b2.reshape(1, 32),
        a1, num_w1, num_b1.reshape(1, 32), num_w2, num_b2.reshape(1, 16),
        attr_w, attr_b.reshape(1, 32),
    ]
    rest = pl.pallas_call(
        _tc1_body,
        grid=(GRID,),
        in_specs=tc1_row_in + [_full_spec(a.shape) for a in tc1_bcast],
        out_specs=pl.BlockSpec((BLK, 128), lambda i: (i, 0)),
        out_shape=jax.ShapeDtypeStruct((B, 128), f32),
    )(
        pluscode_indices.astype(jnp.int32).T,
        temporal_features.T,
        star_idx.astype(jnp.int32).reshape(1, B),
        numerical_features.T,
        *tc1_bcast,
    )

    table3b, rest = jax.lax.optimization_barrier((table3, rest))
    cat_emb = _sc_gather(idx2d, table3b)

    tc2_bcast = [
        fus_w1, fus_b1.reshape(1, 512), fus_ln1_g.reshape(1, 512), fus_ln1_b.reshape(1, 512),
        fus_w2, fus_b2.reshape(1, 256), fus_ln2_g.reshape(1, 256), fus_ln2_b.reshape(1, 256),
    ]
    out = pl.pallas_call(
        _tc2_body,
        grid=(GRID,),
        in_specs=[_row_spec(CAT_D), _row_spec(128)]
        + [_full_spec(a.shape) for a in tc2_bcast],
        out_specs=pl.BlockSpec((BLK, 256), lambda i: (i, 0)),
        out_shape=jax.ShapeDtypeStruct((B, 256), f32),
    )(cat_emb, rest, *tc2_bcast)
    return out


# speedup vs baseline: 1.0341x; 1.0013x over previous
import jax
import jax.numpy as jnp
from jax import lax
from jax.experimental import pallas as pl
from jax.experimental.pallas import tpu as pltpu
from jax.experimental.pallas import tpu_sc as plsc

B = 16384
NC, NS = 2, 16
NW = NC * NS
CAT_D = 64
ROWS_PER_W = B // NW
GROUP = 32

BLK = 1024
GRID = B // BLK


def _sc_gather_body(idx_hbm, table_hbm, out_hbm, idx_v, rows_v, sem):
    wid = lax.axis_index("s") * NC + lax.axis_index("c")
    pltpu.sync_copy(idx_hbm.at[wid], idx_v)

    def issue(g):
        base = g * GROUP
        for k in range(GROUP // 16):
            ids16 = idx_v[pl.ds(base + k * 16, 16)]
            t16 = lax.shift_right_logical(ids16, 3)
            r16 = lax.bitwise_and(ids16, 7)
            for j in range(16):
                pltpu.async_copy(
                    table_hbm.at[t16[j], r16[j]],
                    rows_v.at[base + k * 16 + j], sem)

    def drain():
        for j in range(GROUP):
            pltpu.make_async_copy(table_hbm.at[0, 0], rows_v.at[0], sem).wait()

    issue(0)

    def group(g, carry):
        issue(g)
        drain()
        return carry

    lax.fori_loop(1, ROWS_PER_W // GROUP, group, 0)
    drain()
    pltpu.sync_copy(rows_v, out_hbm.at[pl.ds(wid * ROWS_PER_W, ROWS_PER_W)])


def _sc_gather(idx2d, table3):
    gather = pl.kernel(
        _sc_gather_body,
        out_type=jax.ShapeDtypeStruct((B, CAT_D), jnp.float32),
        mesh=plsc.VectorSubcoreMesh(
            core_axis_name="c", subcore_axis_name="s",
            num_cores=NC, num_subcores=NS,
        ),
        scratch_types=[
            pltpu.VMEM((ROWS_PER_W,), jnp.int32),
            pltpu.VMEM((ROWS_PER_W, CAT_D), jnp.float32),
            pltpu.SemaphoreType.DMA,
        ],
    )
    return gather(idx2d, table3)


def _prep_body(charp, pc_w1, pos, pc_b1, starp, attr_w, mbig, c0, a1):
    cp = charp[...]
    w1 = pc_w1[...]
    parts = [
        jnp.dot(cp, w1[32 * j:32 * (j + 1), :], preferred_element_type=jnp.float32)
        for j in range(10)
    ]
    mbig[...] = jnp.concatenate(parts, axis=0)
    acc = pc_b1[...]
    for j in range(10):
        acc = acc + jnp.dot(
            pos[j:j + 1, :], w1[32 * j:32 * (j + 1), :],
            preferred_element_type=jnp.float32,
        )
    c0[...] = acc
    a1[...] = jnp.dot(starp[...], attr_w[:16, :], preferred_element_type=jnp.float32)


def _ln(x, g, b, eps=1e-5):
    m = jnp.mean(x, axis=-1, keepdims=True)
    v = jnp.var(x, axis=-1, keepdims=True)
    return (x - m) / jnp.sqrt(v + eps) * g + b


_TDN = (((0,), (0,)), ((), ()))


def _tdot(lhs_t, rhs):
    return lax.dot_general(lhs_t, rhs, _TDN, preferred_element_type=jnp.float32)


def _tc1_body(pc_idx_t, temporal_t, star1_t, numerical_t,
              mbig, c0, pc_ln1_g, pc_ln1_b, pc_w2, pc_b2, pc_ln2_g, pc_ln2_b,
              tmp_w1, tmp_b1, tmp_w2, tmp_b2,
              a1, num_w1, num_b1, num_w2, num_b2, attr_w, attr_b,
              out_ref):
    f32 = jnp.float32
    idx_t = pc_idx_t[...]
    iota32 = lax.broadcasted_iota(jnp.int32, (32, BLK), 0)
    ohs_t = [(idx_t[j:j + 1, :] == iota32).astype(f32) for j in range(10)]
    oh_t = jnp.concatenate(ohs_t, axis=0)
    h = _tdot(oh_t, mbig[...]) + c0[...]
    h = jax.nn.gelu(_ln(h, pc_ln1_g[...], pc_ln1_b[...]))
    sp = _ln(jnp.dot(h, pc_w2[...], preferred_element_type=f32) + pc_b2[...],
             pc_ln2_g[...], pc_ln2_b[...])

    t = jax.nn.relu(_tdot(temporal_t[...], tmp_w1[...]) + tmp_b1[...])
    tmp_emb = jnp.dot(t, tmp_w2[...], preferred_element_type=f32) + tmp_b2[...]

    iota16 = lax.broadcasted_iota(jnp.int32, (16, BLK), 0)
    oh_star_t = (star1_t[...] == iota16).astype(f32)
    n = jax.nn.relu(_tdot(numerical_t[...], num_w1[...]) + num_b1[...])
    n = jnp.dot(n, num_w2[...], preferred_element_type=f32) + num_b2[...]
    attr = (_tdot(oh_star_t, a1[...])
            + jnp.dot(n, attr_w[16:32, :], preferred_element_type=f32)
            + attr_b[...])
    out_ref[...] = jnp.concatenate([sp, tmp_emb, attr], axis=-1)


def _tc2_body(cat_emb, rest,
              fus_w1, fus_b1, fus_ln1_g, fus_ln1_b,
              fus_w2, fus_b2, fus_ln2_g, fus_ln2_b,
              out_ref):
    f32 = jnp.float32
    comb = jnp.concatenate([cat_emb[...], rest[...]], axis=-1)
    z = jnp.dot(comb, fus_w1[...], preferred_element_type=f32) + fus_b1[...]
    z = jax.nn.gelu(_ln(z, fus_ln1_g[...], fus_ln1_b[...]))
    o = jnp.dot(z, fus_w2[...], preferred_element_type=f32) + fus_b2[...]
    out_ref[...] = _ln(o, fus_ln2_g[...], fus_ln2_b[...])


def _row_spec(d):
    return pl.BlockSpec((BLK, d), lambda i: (i, 0))


def _full_spec(shape):
    return pl.BlockSpec(shape, lambda i: tuple(0 for _ in shape))


def kernel(category_ids, pluscode_indices, temporal_features, star_idx,
           numerical_features, cat_table, char_table, pos_table,
           pc_w1, pc_b1, pc_ln1_g, pc_ln1_b, pc_w2, pc_b2, pc_ln2_g, pc_ln2_b,
           tmp_w1, tmp_b1, tmp_w2, tmp_b2,
           star_table, num_w1, num_b1, num_w2, num_b2, attr_w, attr_b,
           fus_w1, fus_b1, fus_ln1_g, fus_ln1_b, fus_w2, fus_b2,
           fus_ln2_g, fus_ln2_b):
    f32 = jnp.float32

    ids32 = category_ids.astype(jnp.int32)
    idx2d = ids32.reshape(NW, ROWS_PER_W)
    table3 = cat_table.reshape(cat_table.shape[0] // 8, 8, CAT_D)

    charp = jnp.zeros((32, 32), f32).at[:22, :].set(char_table)
    starp = jnp.zeros((16, 16), f32).at[:11, :].set(star_table)

    mbig, c0, a1 = pl.pallas_call(
        _prep_body,
        out_shape=[
            jax.ShapeDtypeStruct((320, 128), f32),
            jax.ShapeDtypeStruct((1, 128), f32),
            jax.ShapeDtypeStruct((16, 32), f32),
        ],
    )(charp, pc_w1, pos_table, pc_b1.reshape(1, 128), starp, attr_w)

    tc1_row_in = [
        pl.BlockSpec((10, BLK), lambda i: (0, i)),
        pl.BlockSpec((6, BLK), lambda i: (0, i)),
        pl.BlockSpec((1, BLK), lambda i: (0, i)),
        pl.BlockSpec((3, BLK), lambda i: (0, i)),
    ]
    tc1_bcast = [
        mbig, c0, pc_ln1_g.reshape(1, 128), pc_ln1_b.reshape(1, 128),
        pc_w2, pc_b2.reshape(1, 64), pc_ln2_g.reshape(1, 64), pc_ln2_b.reshape(1, 64),
        tmp_w1, tmp_b1.reshape(1, 64), tmp_w2, tmp_b2.reshape(1, 32),
        a1, num_w1, num_b1.reshape(1, 32), num_w2, num_b2.reshape(1, 16),
        attr_w, attr_b.reshape(1, 32),
    ]
    rest = pl.pallas_call(
        _tc1_body,
        grid=(GRID,),
        in_specs=tc1_row_in + [_full_spec(a.shape) for a in tc1_bcast],
        out_specs=pl.BlockSpec((BLK, 128), lambda i: (i, 0)),
        out_shape=jax.ShapeDtypeStruct((B, 128), f32),
    )(
        pluscode_indices.astype(jnp.int32).T,
        temporal_features.T,
        star_idx.astype(jnp.int32).reshape(1, B),
        numerical_features.T,
        *tc1_bcast,
    )

    table3b, rest = jax.lax.optimization_barrier((table3, rest))
    cat_emb = _sc_gather(idx2d, table3b)

    tc2_bcast = [
        fus_w1, fus_b1.reshape(1, 512), fus_ln1_g.reshape(1, 512), fus_ln1_b.reshape(1, 512),
        fus_w2, fus_b2.reshape(1, 256), fus_ln2_g.reshape(1, 256), fus_ln2_b.reshape(1, 256),
    ]
    out = pl.pallas_call(
        _tc2_body,
        grid=(GRID,),
        in_specs=[_row_spec(CAT_D), _row_spec(128)]
        + [_full_spec(a.shape) for a in tc2_bcast],
        out_specs=pl.BlockSpec((BLK, 256), lambda i: (i, 0)),
        out_shape=jax.ShapeDtypeStruct((B, 256), f32),
    )(cat_emb, rest, *tc2_bcast)
    return out
